# baseline (speedup 1.0000x reference)
import jax
import jax.numpy as jnp
from jax import lax
from jax.experimental import pallas as pl
from jax.experimental.pallas import tpu as pltpu
from jax.experimental.pallas import tpu_sc as plsc

_V = 1000000
_BATCH = 16384
_K = 16
_NC = 2
_NS = 16
_NW = _NC * _NS
_BPW = _BATCH // _NW
_CHUNK = 128
_NCHUNK = _BPW // _CHUNK
_NFCH = _BPW * _K // _CHUNK
_GROUPS = _BPW // _K


def _mf_sc_kernel(uidx_hbm, vidx_hbm, w_hbm, h_hbm, out_hbm,
                  uidx_v, vidx_v, fidx_v, gidx_v, urows_v, vrows_v, out_v,
                  sem):
    wid = lax.axis_index("s") * _NC + lax.axis_index("c")
    base = wid * _BPW

    pltpu.sync_copy(uidx_hbm.at[pl.ds(wid * _NCHUNK, _NCHUNK)], uidx_v)
    pltpu.sync_copy(vidx_hbm.at[pl.ds(wid * _NCHUNK, _NCHUNK)], vidx_v)

    def idx_body(c, carry):
        for j in range(_NCHUNK):
            for s in range(_CHUNK // _K):
                ids = uidx_v[j, pl.ds(s * _K, _K)]
                gds = vidx_v[j, pl.ds(s * _K, _K)]
                fidx_v[c * _NCHUNK + j, pl.ds(s * _K, _K)] = ids + c * _V
                gidx_v[c * _NCHUNK + j, pl.ds(s * _K, _K)] = gds + c * _V
        return carry

    lax.fori_loop(0, _K, idx_body, 0)

    copies = []
    for j in range(_NFCH):
        copies.append(pltpu.async_copy(
            w_hbm.at[fidx_v.at[j]], urows_v.at[pl.ds(j * _CHUNK, _CHUNK)],
            sem))
        copies.append(pltpu.async_copy(
            h_hbm.at[gidx_v.at[j]], vrows_v.at[pl.ds(j * _CHUNK, _CHUNK)],
            sem))
    for cp in copies:
        cp.wait()

    def group_body(g, carry):
        acc = jnp.zeros((_K,), jnp.float32)
        for c in range(_K):
            off = c * _BPW + g * _K
            acc = acc + urows_v[pl.ds(off, _K)] * vrows_v[pl.ds(off, _K)]
        out_v[pl.ds(g * _K, _K)] = 1.0 / (1.0 + jnp.exp(-acc))
        return carry

    lax.fori_loop(0, _GROUPS, group_body, 0)

    pltpu.sync_copy(out_v, out_hbm.at[pl.ds(base, _BPW)])


@jax.jit
def kernel(x, W, H):
    uidx = x[:, 0].reshape(_NW * _NCHUNK, _CHUNK)
    vidx = x[:, 1].reshape(_NW * _NCHUNK, _CHUNK)
    wf = W.T.reshape(_V * _K)
    hf = H.T.reshape(_V * _K)
    mesh = plsc.VectorSubcoreMesh(core_axis_name="c", subcore_axis_name="s")
    run = pl.kernel(
        _mf_sc_kernel,
        out_type=jax.ShapeDtypeStruct((_BATCH,), jnp.float32),
        mesh=mesh,
        scratch_types=[
            pltpu.VMEM((_NCHUNK, _CHUNK), jnp.int32),
            pltpu.VMEM((_NCHUNK, _CHUNK), jnp.int32),
            pltpu.VMEM((_NFCH, _CHUNK), jnp.int32),
            pltpu.VMEM((_NFCH, _CHUNK), jnp.int32),
            pltpu.VMEM((_BPW * _K,), jnp.float32),
            pltpu.VMEM((_BPW * _K,), jnp.float32),
            pltpu.VMEM((_BPW,), jnp.float32),
            pltpu.SemaphoreType.DMA,
        ],
        compiler_params=pltpu.CompilerParams(
            needs_layout_passes=False, use_tc_tiling_on_sc=False),
    )
    return run(uidx, vidx, wf, hf)


# speedup vs baseline: 20.7075x; 20.7075x over previous
import jax
import jax.numpy as jnp
from jax import lax
from jax.experimental import pallas as pl
from jax.experimental.pallas import tpu as pltpu
from jax.experimental.pallas import tpu_sc as plsc

_V = 1000000
_VT = 7813
_VP = _VT * 128
_BATCH = 16384
_K = 16
_NC = 2
_NS = 16
_NW = _NC * _NS
_BPW = _BATCH // _NW
_CHUNK = 128
_NCHUNK = _BPW // _CHUNK
_NFCH = _BPW * _K // _CHUNK
_GROUPS = _BPW // _K


def _mf_sc_kernel(uidx_hbm, vidx_hbm, w_hbm, h_hbm, out_hbm,
                  uidx_v, vidx_v, fidx_v, gidx_v, urows_v, vrows_v, out_v,
                  sem):
    wid = lax.axis_index("s") * _NC + lax.axis_index("c")
    base = wid * _BPW

    pltpu.sync_copy(uidx_hbm.at[pl.ds(wid * _NCHUNK, _NCHUNK)], uidx_v)
    pltpu.sync_copy(vidx_hbm.at[pl.ds(wid * _NCHUNK, _NCHUNK)], vidx_v)

    def idx_body(c, carry):
        cbase = (c // 8) * (_VT * 1024) + (c % 8) * 128
        for j in range(_NCHUNK):
            for s in range(_CHUNK // _K):
                ids = uidx_v[j, pl.ds(s * _K, _K)]
                gds = vidx_v[j, pl.ds(s * _K, _K)]
                fidx_v[c * _NCHUNK + j, pl.ds(s * _K, _K)] = (
                    cbase + (ids >> 7) * 1024 + (ids & 127))
                gidx_v[c * _NCHUNK + j, pl.ds(s * _K, _K)] = (
                    cbase + (gds >> 7) * 1024 + (gds & 127))
        return carry

    lax.fori_loop(0, _K, idx_body, 0)

    copies = []
    for j in range(_NFCH):
        copies.append(pltpu.async_copy(
            w_hbm.at[fidx_v.at[j]], urows_v.at[pl.ds(j * _CHUNK, _CHUNK)],
            sem))
        copies.append(pltpu.async_copy(
            h_hbm.at[gidx_v.at[j]], vrows_v.at[pl.ds(j * _CHUNK, _CHUNK)],
            sem))
    for cp in copies:
        cp.wait()

    def group_body(g, carry):
        acc = jnp.zeros((_K,), jnp.float32)
        for c in range(_K):
            off = c * _BPW + g * _K
            acc = acc + urows_v[pl.ds(off, _K)] * vrows_v[pl.ds(off, _K)]
        out_v[pl.ds(g * _K, _K)] = 1.0 / (1.0 + jnp.exp(-acc))
        return carry

    lax.fori_loop(0, _GROUPS, group_body, 0)

    pltpu.sync_copy(out_v, out_hbm.at[pl.ds(base, _BPW)])


def _tiled_flat(t):
    tp = jnp.pad(t, ((0, _VP - _V), (0, 0)))
    return (tp.reshape(_VT, 128, 2, 8)
            .transpose(2, 0, 3, 1)
            .reshape(2 * _VT * 8 * 128))


@jax.jit
def kernel(x, W, H):
    uidx = x[:, 0].reshape(_NW * _NCHUNK, _CHUNK)
    vidx = x[:, 1].reshape(_NW * _NCHUNK, _CHUNK)
    wf = _tiled_flat(W)
    hf = _tiled_flat(H)
    mesh = plsc.VectorSubcoreMesh(core_axis_name="c", subcore_axis_name="s")
    run = pl.kernel(
        _mf_sc_kernel,
        out_type=jax.ShapeDtypeStruct((_BATCH,), jnp.float32),
        mesh=mesh,
        scratch_types=[
            pltpu.VMEM((_NCHUNK, _CHUNK), jnp.int32),
            pltpu.VMEM((_NCHUNK, _CHUNK), jnp.int32),
            pltpu.VMEM((_NFCH, _CHUNK), jnp.int32),
            pltpu.VMEM((_NFCH, _CHUNK), jnp.int32),
            pltpu.VMEM((_BPW * _K,), jnp.float32),
            pltpu.VMEM((_BPW * _K,), jnp.float32),
            pltpu.VMEM((_BPW,), jnp.float32),
            pltpu.SemaphoreType.DMA,
        ],
        compiler_params=pltpu.CompilerParams(
            needs_layout_passes=False, use_tc_tiling_on_sc=False),
    )
    return run(uidx, vidx, wf, hf)
